# baseline (speedup 1.0000x reference)
import functools

import jax
import jax.numpy as jnp
from jax import lax
from jax.experimental import pallas as pl
from jax.experimental.pallas import tpu as pltpu
from jax.experimental.pallas import tpu_sc as plsc

B, HEADS, L, DIM, CODES = 4, 16, 4096, 64, 128
TL = 2048
HP = HEADS // 2
NTOK = B * HEADS * L

NC, NS = 2, 16
NW = NC * NS
TPW = NTOK // NW
CH = 256
NCHUNK = TPW // CH


def _idx_body(xt_ref, w_ref, idx_ref):
    a = xt_ref[...].reshape(2 * DIM, TL)
    sim = jax.lax.dot_general(
        a, w_ref[...],
        dimension_numbers=(((0,), (0,)), ((), ())),
        preferred_element_type=jnp.float32,
    )
    iota_f = jax.lax.broadcasted_iota(
        jnp.int32, (TL, CODES), 1).astype(jnp.float32)

    def half(sim_h, out):
        m = jnp.max(sim_h, axis=-1, keepdims=True)
        masked = jnp.where(sim_h == m, iota_f, float(CODES))
        idxf = jnp.min(masked, axis=-1)
        out[...] = idxf.astype(jnp.int32).reshape(8, TL // 8)

    half(sim[:, :CODES], idx_ref.at[0])
    half(sim[:, CODES:], idx_ref.at[1])


@jax.jit
def _idx_call(x, c):
    cT = jnp.swapaxes(c, 1, 2)
    z = jnp.zeros((HP, DIM, CODES), jnp.float32)
    w = jnp.concatenate([
        jnp.concatenate([cT[0::2], z], axis=-1),
        jnp.concatenate([z, cT[1::2]], axis=-1),
    ], axis=1)
    xt = jnp.transpose(x, (0, 1, 3, 2))
    grid = (HP, B, L // TL)
    idx = pl.pallas_call(
        _idx_body,
        grid=grid,
        in_specs=[
            pl.BlockSpec((None, 2, DIM, TL), lambda g, b, j: (b, g, 0, j)),
            pl.BlockSpec((None, 2 * DIM, 2 * CODES), lambda g, b, j: (g, 0, 0)),
        ],
        out_specs=pl.BlockSpec((None, 2, None, 8, TL // 8),
                               lambda g, b, j: (b, g, j, 0, 0)),
        out_shape=jax.ShapeDtypeStruct((B, HEADS, L // TL, 8, TL // 8),
                                       jnp.int32),
        compiler_params=pltpu.CompilerParams(
            dimension_semantics=("parallel", "parallel", "arbitrary")),
    )(xt, w)
    return idx.reshape(NTOK)


def _make_sc_scatter():
    mesh = plsc.VectorSubcoreMesh(core_axis_name="c", subcore_axis_name="s")

    @functools.partial(
        pl.kernel, mesh=mesh,
        out_type=jax.ShapeDtypeStruct((NTOK, CODES), jnp.float32),
        scratch_types=[
            pltpu.VMEM((CH,), jnp.int32),
            pltpu.VMEM((CH, CODES), jnp.float32),
        ],
        compiler_params=pltpu.CompilerParams(needs_layout_passes=False),
    )
    def sc_scatter(idx_hbm, out_hbm, idx_v, buf):
        wid = lax.axis_index("s") * NC + lax.axis_index("c")
        base = wid * TPW

        lane = lax.iota(jnp.int32, 16)
        ones = jnp.full((16,), 1.0, jnp.float32)
        zeros = jnp.zeros((16,), jnp.float32)

        def zero_body(i, _):
            buf[i, pl.ds(0, 16)] = zeros
            return _

        def zero_row(i, _):
            for j in range(CODES // 16):
                buf[i, pl.ds(j * 16, 16)] = zeros
            return _

        lax.fori_loop(0, CH, zero_row, 0)

        def chunk_body(ci, _):
            start = base + ci * CH
            pltpu.sync_copy(idx_hbm.at[pl.ds(start, CH)], idx_v)
            for i in range(CH // 16):
                code = idx_v[pl.ds(i * 16, 16)]
                tok = lane + i * 16
                plsc.store_scatter(buf, [tok, code], ones)
            pltpu.sync_copy(buf, out_hbm.at[pl.ds(start, CH)])
            for i in range(CH // 16):
                code = idx_v[pl.ds(i * 16, 16)]
                tok = lane + i * 16
                plsc.store_scatter(buf, [tok, code], zeros)
            return _

        lax.fori_loop(0, NCHUNK, chunk_body, 0)

    return sc_scatter


_sc_scatter = _make_sc_scatter()


def kernel(x, c):
    idx = _idx_call(x, c)
    onehot = _sc_scatter(idx)
    return (onehot.reshape(B, HEADS, L, CODES), c)


# speedup vs baseline: 1.9370x; 1.9370x over previous
import functools

import jax
import jax.numpy as jnp
from jax.experimental import pallas as pl
from jax.experimental.pallas import tpu as pltpu

B, HEADS, L, DIM, CODES = 4, 16, 4096, 64, 128
TL = 4096
HP = HEADS // 2


def _onehot_half(sim, iota_f, out_ref):
    m = jnp.max(sim, axis=-1, keepdims=True)
    masked = jnp.where(sim == m, iota_f, float(CODES))
    idxf = jnp.min(masked, axis=-1, keepdims=True)
    out_ref[...] = jnp.where(iota_f == idxf, 1.0, 0.0)


def _fused_body(xt_ref, w_ref, out_ref):
    a = xt_ref[...].reshape(2 * DIM, TL)
    sim = jax.lax.dot_general(
        a, w_ref[...],
        dimension_numbers=(((0,), (0,)), ((), ())),
        preferred_element_type=jnp.float32,
    )
    iota_f = jax.lax.broadcasted_iota(
        jnp.int32, (TL, CODES), 1).astype(jnp.float32)
    _onehot_half(sim[:, :CODES], iota_f, out_ref.at[0])
    _onehot_half(sim[:, CODES:], iota_f, out_ref.at[1])


@functools.partial(jax.jit, static_argnames=("interpret",))
def _fused_call(x, c, interpret=False):
    cT = jnp.swapaxes(c, 1, 2)
    z = jnp.zeros((HP, DIM, CODES), jnp.float32)
    w = jnp.concatenate([
        jnp.concatenate([cT[0::2], z], axis=-1),
        jnp.concatenate([z, cT[1::2]], axis=-1),
    ], axis=1)
    xt = jnp.transpose(x, (0, 1, 3, 2))
    grid = (HP, B, L // TL)
    out = pl.pallas_call(
        _fused_body,
        grid=grid,
        in_specs=[
            pl.BlockSpec((None, 2, DIM, TL), lambda g, b, j: (b, g, 0, j)),
            pl.BlockSpec((None, 2 * DIM, 2 * CODES), lambda g, b, j: (g, 0, 0)),
        ],
        out_specs=pl.BlockSpec((None, 2, TL, CODES),
                               lambda g, b, j: (b, g, j, 0)),
        out_shape=jax.ShapeDtypeStruct((B, HEADS, L, CODES), jnp.float32),
        compiler_params=pltpu.CompilerParams(
            dimension_semantics=("parallel", "parallel", "arbitrary")),
        interpret=interpret,
    )(xt, w)
    return out


def kernel(x, c):
    onehot = _fused_call(x, c)
    return (onehot, c)
